# baseline (speedup 1.0000x reference)
import functools

import jax
import jax.numpy as jnp
from jax import lax
from jax.experimental import pallas as pl
from jax.experimental.pallas import tpu as pltpu
from jax.experimental.pallas import tpu_sc as plsc

N_TOKENS = 1000
B, T = 4096, 20
NW = 32
ROWS_W = B // NW
CB = 2
TOK = CB * T
NCHUNK = ROWS_W // CB
TOK_W = ROWS_W * T


@functools.cache
def _build():
    mesh = plsc.VectorSubcoreMesh(core_axis_name="c", subcore_axis_name="s")

    @functools.partial(
        pl.kernel,
        mesh=mesh,
        compiler_params=pltpu.CompilerParams(needs_layout_passes=False),
        out_type=jax.ShapeDtypeStruct((B, T, N_TOKENS), jnp.int32),
        scratch_types=[
            pltpu.VMEM((TOK_W + 16,), jnp.int32),
            pltpu.VMEM((2, CB, T, N_TOKENS), jnp.int32),
            pltpu.SemaphoreType.DMA,
            pltpu.SemaphoreType.DMA,
        ],
    )
    def _sc_onehot(x_hbm, z_hbm, out_hbm, idx_v, bufs, sem0, sem1):
        sems = (sem0, sem1)
        wid = lax.axis_index("s") * 2 + lax.axis_index("c")
        tok0 = wid * TOK_W
        row0 = wid * ROWS_W

        pltpu.sync_copy(x_hbm.at[pl.ds(tok0, TOK_W)], idx_v.at[pl.ds(0, TOK_W)])
        pltpu.sync_copy(z_hbm, bufs.at[0])
        pltpu.sync_copy(z_hbm, bufs.at[1])

        lane = lax.iota(jnp.int32, 16)
        ones = jnp.ones((16,), jnp.int32)
        zeros = jnp.zeros((16,), jnp.int32)

        def scatter(buf, i, val):
            base = i * TOK
            for j in range(3):
                t = lane + j * 16
                col = idx_v[pl.ds(base + j * 16, 16)]
                b0 = jnp.minimum(t // T, CB - 1)
                r = t % T
                if (j + 1) * 16 <= TOK:
                    plsc.store_scatter(buf, [b0, r, col], val)
                else:
                    plsc.store_scatter(buf, [b0, r, col], val,
                                       mask=lane < (TOK - j * 16))

        def out_slice(i):
            return out_hbm.at[pl.ds(row0 + i * CB, CB)]

        for b in range(2):
            scatter(bufs.at[b], b, ones)
            pltpu.make_async_copy(bufs.at[b], out_slice(b), sems[b]).start()

        def g_body(g, carry):
            for b in range(2):
                i = g * 2 + b
                pltpu.make_async_copy(bufs.at[b], out_slice(0), sems[b]).wait()
                scatter(bufs.at[b], i - 2, zeros)
                scatter(bufs.at[b], i, ones)
                pltpu.make_async_copy(bufs.at[b], out_slice(i), sems[b]).start()
            return carry

        lax.fori_loop(1, NCHUNK // 2, g_body, 0)

        for b in range(2):
            pltpu.make_async_copy(bufs.at[b], out_slice(0), sems[b]).wait()

    return _sc_onehot


def kernel(x):
    xf = x.reshape(-1).astype(jnp.int32)
    z = jnp.zeros((CB, T, N_TOKENS), jnp.int32)
    out = _build()(xf, z)
    return out.astype(x.dtype)
